# baseline (speedup 1.0000x reference)
import functools

import jax
import jax.numpy as jnp
from jax import lax
from jax.experimental import pallas as pl
from jax.experimental.pallas import tpu as pltpu
from jax.experimental.pallas import tpu_sc as plsc

EMB = 64
BATCH = 4096
NROWS = 100000
L = 16
NC = 2
NS = 16
NW = NC * NS
PIECE = 4992
NPIECE = 10
HALF0 = NPIECE * PIECE
TAIL = NROWS - 2 * HALF0
NEDGE = 2 * NPIECE + 2


def _sc_gather(u_t, f_t, ub1, fb1, usort, upos, fsort, fpos, ubnd, fbnd):
    mesh = plsc.VectorSubcoreMesh(core_axis_name="c", subcore_axis_name="s")

    @functools.partial(
        pl.kernel,
        mesh=mesh,
        compiler_params=pltpu.CompilerParams(needs_layout_passes=False),
        out_type=(
            jax.ShapeDtypeStruct((16, 8, BATCH), jnp.float32),
            jax.ShapeDtypeStruct((16, 8, BATCH), jnp.float32),
            jax.ShapeDtypeStruct((2, 8, BATCH), jnp.float32),
            jax.ShapeDtypeStruct((2, 8, BATCH), jnp.float32),
        ),
        scratch_types=[
            pltpu.VMEM((BATCH,), jnp.int32),
            pltpu.VMEM((BATCH,), jnp.int32),
            pltpu.VMEM((32,), jnp.int32),
            pltpu.VMEM((8, PIECE), jnp.float32),
            pltpu.VMEM((8, PIECE), jnp.float32),
            pltpu.VMEM((8, BATCH), jnp.float32),
            pltpu.VMEM((PIECE,), jnp.float32),
            pltpu.VMEM((TAIL,), jnp.float32),
            pltpu.SemaphoreType.DMA,
            pltpu.SemaphoreType.DMA,
        ],
    )
    def k(ut_h, ft_h, ub_h, fb_h, us_h, up_h, fs_h, fp_h, ubnd_h, fbnd_h,
          uout, fout, ubg_out, fbg_out,
          ids_v, pos_v, bnd_v, pa_v, pb_v, stage_v, b1d_v, tail_v,
          sem_a, sem_b):
        wid = lax.axis_index("s") * NC + lax.axis_index("c")
        tbl = wid // 16
        rest = wid % 16
        tr = rest // 2
        h = rest % 2
        base = h * HALF0
        lane = jnp.arange(L, dtype=jnp.int32)
        zf = jnp.zeros((L,), jnp.float32)

        def zero_stage(c, _):
            for q in range(4):
                stage_v[(c * 4 + q) // 256,
                        pl.ds(((c * 4 + q) % 256) * L, L)] = zf
            return 0

        def run(tab_h, bias_h, sid_h, spos_h, bnd_h, out_ref, bout_ref):
            pltpu.sync_copy(sid_h, ids_v)
            pltpu.sync_copy(spos_h, pos_v)
            pltpu.sync_copy(bnd_h, bnd_v)
            lax.fori_loop(0, (8 * BATCH) // L // 4, zero_stage, 0)

            bufs = (pa_v, pb_v)
            sems = (sem_a, sem_b)

            def bound(j):
                jv = jnp.full((L,), 0, jnp.int32) + j
                return jnp.max(plsc.load_gather(bnd_v, [jv]))

            def start_piece(p):
                lo = base + p * PIECE
                return pltpu.async_copy(
                    tab_h.at[pl.ds(tr * 8, 8), pl.ds(lo, PIECE)],
                    bufs[p % 2], sems[p % 2])

            def gather_range(load_fn, a, b):
                def body(c, _):
                    lpos = c * L + lane
                    mv = (lpos >= a) & (lpos < b)
                    sidc = ids_v[pl.ds(c * L, L)]
                    posc = pos_v[pl.ds(c * L, L)]
                    load_fn(sidc, posc, mv)
                    return 0

                return lax.fori_loop(a // L, (b + L - 1) // L, body, 0)

            def emb_loader(buf, lo):
                def fn(sidc, posc, mv):
                    lidxc = sidc - lo
                    for dd in range(8):
                        dvec = jnp.full((L,), dd, jnp.int32)
                        v = plsc.load_gather(buf, [dvec, lidxc], mask=mv)
                        plsc.store_scatter(stage_v, [dvec, posc], v, mask=mv)
                return fn

            cp = start_piece(0)
            for p in range(NPIECE):
                jidx = h * NPIECE + p
                a = bound(jidx)
                b = bound(jidx + 1)
                lo = base + p * PIECE
                cp.wait()
                if p + 1 < NPIECE:
                    cp = start_piece(p + 1)
                gather_range(emb_loader(bufs[p % 2], lo), a, b)

            @pl.when(h == 1)
            def _():
                lo = 2 * HALF0
                a = bound(2 * NPIECE)
                b = bound(2 * NPIECE + 1)
                for dd in range(8):
                    pltpu.sync_copy(
                        tab_h.at[tr * 8 + dd, pl.ds(lo, TAIL)], tail_v)
                    dvec = jnp.full((L,), dd, jnp.int32)

                    def fn(sidc, posc, mv, dvec=dvec):
                        v = plsc.load_gather(tail_v, [sidc - lo], mask=mv)
                        plsc.store_scatter(stage_v, [dvec, posc], v, mask=mv)

                    gather_range(fn, a, b)

            pltpu.sync_copy(stage_v, out_ref.at[tr * 2 + h])

            @pl.when(tr == 0)
            def _():
                def zrow(c, _):
                    for q in range(4):
                        stage_v[0, pl.ds((c * 4 + q) * L, L)] = zf
                    return 0

                lax.fori_loop(0, BATCH // L // 4, zrow, 0)
                d0 = jnp.zeros((L,), jnp.int32)
                for p in range(NPIECE):
                    jidx = h * NPIECE + p
                    a = bound(jidx)
                    b = bound(jidx + 1)
                    lo = base + p * PIECE
                    pltpu.sync_copy(bias_h.at[pl.ds(lo, PIECE)], b1d_v)

                    def bfn(sidc, posc, mv):
                        v = plsc.load_gather(b1d_v, [sidc - lo], mask=mv)
                        plsc.store_scatter(stage_v, [d0, posc], v, mask=mv)

                    gather_range(bfn, a, b)

                @pl.when(h == 1)
                def _():
                    lo = 2 * HALF0
                    a = bound(2 * NPIECE)
                    b = bound(2 * NPIECE + 1)
                    pltpu.sync_copy(bias_h.at[pl.ds(lo, TAIL)], tail_v)

                    def bfn2(sidc, posc, mv):
                        v = plsc.load_gather(tail_v, [sidc - lo], mask=mv)
                        plsc.store_scatter(stage_v, [d0, posc], v, mask=mv)

                    gather_range(bfn2, a, b)

                pltpu.sync_copy(stage_v, bout_ref.at[h])

        @pl.when(tbl == 0)
        def _():
            run(ut_h, ub_h, us_h, up_h, ubnd_h, uout, ubg_out)

        @pl.when(tbl == 1)
        def _():
            run(ft_h, fb_h, fs_h, fp_h, fbnd_h, fout, fbg_out)

    return k(u_t, f_t, ub1, fb1, usort, upos, fsort, fpos, ubnd, fbnd)


def _tc_mlp(ublk, fblk, ubg, fbg, w1r, b1r, w2, b2r, w3r, b3r):
    def body(u_ref, f_ref, ub_ref, fb_ref, w1_ref, b1_ref, w2_ref, b2_ref,
             w3_ref, b3_ref, out_ref):
        u4 = u_ref[...].reshape(8, 2, 8, BATCH)
        f4 = f_ref[...].reshape(8, 2, 8, BATCH)
        uv = u4[:, 0] + u4[:, 1]
        fv = f4[:, 0] + f4[:, 1]
        s = jnp.sum(uv * fv)
        ub = (ub_ref[0, 0, :] + ub_ref[1, 0, :]).reshape(BATCH, 1)
        fb = (fb_ref[0, 0, :] + fb_ref[1, 0, :]).reshape(BATCH, 1)
        x = s + ub + fb
        h1 = jnp.maximum(x * w1_ref[...] + b1_ref[...], 0.0)
        h2 = jnp.maximum(
            jnp.dot(h1, w2_ref[...], preferred_element_type=jnp.float32)
            + b2_ref[...], 0.0)
        zz = jnp.sum(h2 * w3_ref[...], axis=1, keepdims=True) + b3_ref[...]
        out_ref[...] = 1.0 / (1.0 + jnp.exp(-zz))

    return pl.pallas_call(
        body,
        out_shape=jax.ShapeDtypeStruct((BATCH, 1), jnp.float32),
    )(ublk, fblk, ubg, fbg, w1r, b1r, w2, b2r, w3r, b3r)


def kernel(inputs, user_emb, user_bias, food_emb, food_bias, W1, b1, W2, b2, W3, b3):
    idx = inputs.astype(jnp.int32)
    uid = idx[:, 0]
    fid = idx[:, 1]
    iota = jnp.arange(BATCH, dtype=jnp.int32)
    usort, upos = lax.sort_key_val(uid, iota)
    fsort, fpos = lax.sort_key_val(fid, iota)
    edges = jnp.minimum(
        jnp.arange(NEDGE, dtype=jnp.int32) * PIECE, NROWS)
    ubnd = jnp.searchsorted(usort, edges).astype(jnp.int32)
    fbnd = jnp.searchsorted(fsort, edges).astype(jnp.int32)
    pad = jnp.zeros((32 - NEDGE,), jnp.int32)
    ubnd = jnp.concatenate([ubnd, pad])
    fbnd = jnp.concatenate([fbnd, pad])
    ublk, fblk, ubg, fbg = _sc_gather(
        user_emb.T, food_emb.T,
        user_bias.reshape(-1), food_bias.reshape(-1),
        usort, upos, fsort, fpos, ubnd, fbnd)
    return _tc_mlp(
        ublk, fblk, ubg, fbg,
        W1.reshape(1, 128), b1.reshape(1, 128),
        W2, b2.reshape(1, 64),
        W3.reshape(1, 64), b3.reshape(1, 1))


# speedup vs baseline: 1.4802x; 1.4802x over previous
import functools

import jax
import jax.numpy as jnp
from jax import lax
from jax.experimental import pallas as pl
from jax.experimental.pallas import tpu as pltpu
from jax.experimental.pallas import tpu_sc as plsc

EMB = 64
BATCH = 4096
NROWS = 100000
L = 16
NC = 2
NS = 16
NW = NC * NS
NCHUNK = BATCH // L


def _sc_gather_dot(u_t, f_t, ub1, fb1, uid, fid):
    mesh = plsc.VectorSubcoreMesh(core_axis_name="c", subcore_axis_name="s")

    @functools.partial(
        pl.kernel,
        mesh=mesh,
        compiler_params=pltpu.CompilerParams(needs_layout_passes=False),
        out_type=(
            jax.ShapeDtypeStruct((NW, L), jnp.float32),
            jax.ShapeDtypeStruct((BATCH,), jnp.float32),
            jax.ShapeDtypeStruct((BATCH,), jnp.float32),
        ),
        scratch_types=[
            pltpu.VMEM((BATCH,), jnp.int32),
            pltpu.VMEM((BATCH,), jnp.int32),
            pltpu.VMEM((NROWS,), jnp.float32),
            pltpu.VMEM((BATCH,), jnp.float32),
            pltpu.VMEM((L,), jnp.float32),
        ],
    )
    def k(ut_h, ft_h, ub_h, fb_h, uid_h, fid_h,
          part_out, ubg_out, fbg_out,
          uid_v, fid_v, row_v, g_v, acc_v):
        wid = lax.axis_index("s") * NC + lax.axis_index("c")
        pltpu.sync_copy(uid_h, uid_v)
        pltpu.sync_copy(fid_h, fid_v)

        def gather_to_g(c, _):
            for q in range(4):
                sl = pl.ds((c * 4 + q) * L, L)
                g_v[sl] = plsc.load_gather(row_v, [uid_v[sl]])
            return 0

        def gather_f_fma(c, accs):
            a = list(accs)
            for q in range(4):
                sl = pl.ds((c * 4 + q) * L, L)
                a[q] = a[q] + plsc.load_gather(row_v, [fid_v[sl]]) * g_v[sl]
            return tuple(a)

        z = jnp.zeros((L,), jnp.float32)
        accs = (z, z, z, z)
        for p in range(2):
            d = wid + NW * p
            pltpu.sync_copy(ut_h.at[d], row_v)
            lax.fori_loop(0, NCHUNK // 4, gather_to_g, 0)
            pltpu.sync_copy(ft_h.at[d], row_v)
            accs = lax.fori_loop(0, NCHUNK // 4, gather_f_fma, accs)

        a0, a1, a2, a3 = accs
        acc_v[...] = (a0 + a1) + (a2 + a3)
        pltpu.sync_copy(acc_v, part_out.at[wid])

        @pl.when(wid == 0)
        def _():
            pltpu.sync_copy(ub_h, row_v)
            lax.fori_loop(0, NCHUNK // 4, gather_to_g, 0)
            pltpu.sync_copy(g_v, ubg_out)

        @pl.when(wid == 1)
        def _():
            pltpu.sync_copy(fb_h, row_v)

            def gather_fb(c, _):
                for q in range(4):
                    sl = pl.ds((c * 4 + q) * L, L)
                    g_v[sl] = plsc.load_gather(row_v, [fid_v[sl]])
                return 0

            lax.fori_loop(0, NCHUNK // 4, gather_fb, 0)
            pltpu.sync_copy(g_v, fbg_out)

    return k(u_t, f_t, ub1, fb1, uid, fid)


def _tc_mlp(partials, ub, fb, w1r, b1r, w2, b2r, w3r, b3r):
    def body(p_ref, ub_ref, fb_ref, w1_ref, b1_ref, w2_ref, b2_ref,
             w3_ref, b3_ref, out_ref):
        s = jnp.sum(p_ref[...])
        x = s + ub_ref[...] + fb_ref[...]
        h1 = jnp.maximum(x * w1_ref[...] + b1_ref[...], 0.0)
        h2 = jnp.maximum(
            jnp.dot(h1, w2_ref[...], preferred_element_type=jnp.float32)
            + b2_ref[...], 0.0)
        zz = jnp.sum(h2 * w3_ref[...], axis=1, keepdims=True) + b3_ref[...]
        out_ref[...] = 1.0 / (1.0 + jnp.exp(-zz))

    return pl.pallas_call(
        body,
        out_shape=jax.ShapeDtypeStruct((BATCH, 1), jnp.float32),
    )(partials, ub, fb, w1r, b1r, w2, b2r, w3r, b3r)


def kernel(inputs, user_emb, user_bias, food_emb, food_bias, W1, b1, W2, b2, W3, b3):
    idx = inputs.astype(jnp.int32)
    uid = idx[:, 0]
    fid = idx[:, 1]
    partials, ubg, fbg = _sc_gather_dot(
        user_emb.T, food_emb.T,
        user_bias.reshape(-1), food_bias.reshape(-1),
        uid, fid)
    return _tc_mlp(
        partials, ubg.reshape(BATCH, 1), fbg.reshape(BATCH, 1),
        W1.reshape(1, 128), b1.reshape(1, 128),
        W2, b2.reshape(1, 64),
        W3.reshape(1, 64), b3.reshape(1, 1))
